# baseline (speedup 1.0000x reference)
import jax
import jax.numpy as jnp
from jax import lax
from jax.experimental import pallas as pl
from jax.experimental.pallas import tpu as pltpu

_N, _T, _C, _H, _W = 4, 16, 128, 32, 32
_KC, _KT = _C // 2, _T // 2
_HW = _H * _W


def _stats_body(x_ref, ab_ref, m_ref):
    xb = x_ref[0, 0].reshape(_C, _HW)
    s = jnp.sum(xb, axis=1) * (1.0 / _HW)
    mx = jnp.max(xb, axis=1)
    a = ab_ref[0, 0]
    b = ab_ref[0, 1]
    m_ref[0, 0, 0] = (0.5 + a) * s + (0.5 + b) * mx


def _mask_body(m_ref, fc_ref, imp_ref):
    fc = fc_ref[...]
    jlt_c = (lax.broadcasted_iota(jnp.int32, (_C, _C), 1)
             < lax.broadcasted_iota(jnp.int32, (_C, _C), 0))
    jlt_t = (lax.broadcasted_iota(jnp.int32, (_T, _T), 1)
             < lax.broadcasted_iota(jnp.int32, (_T, _T), 0))
    chan = jnp.zeros((_T, _C), jnp.float32)
    ct = jnp.zeros((_C, _T), jnp.float32)
    for n in range(_N):
        mn = m_ref[n, :, 0]
        m2 = jnp.dot(fc, mn, preferred_element_type=jnp.float32)
        vi = m2[:, :, None]
        vj = m2[:, None, :]
        g = (vj > vi) | ((vj == vi) & jlt_c[None])
        cnt = jnp.sum(g.astype(jnp.float32), axis=2)
        chan = jnp.maximum(chan, (cnt < _KC).astype(jnp.float32))
        mt = m2.T
        vi2 = mt[:, :, None]
        vj2 = mt[:, None, :]
        g2 = (vj2 > vi2) | ((vj2 == vi2) & jlt_t[None])
        cnt2 = jnp.sum(g2.astype(jnp.float32), axis=2)
        ct = jnp.maximum(ct, (cnt2 < _KT).astype(jnp.float32))
    ctt = ct.T
    imp = chan + ctt - 2.0 * chan * ctt
    imp_ref[...] = imp[:, None, :]


def _conv3x3_sigmoid(avg_img, max_img, w_ref):
    def _pad1(img):
        zr = jnp.zeros((1, _W), jnp.float32)
        p = jnp.concatenate([zr, img, zr], axis=0)
        zc = jnp.zeros((_H + 2, 1), jnp.float32)
        return jnp.concatenate([zc, p, zc], axis=1)

    pa = _pad1(avg_img)
    pm = _pad1(max_img)
    acc = jnp.zeros((_H, _W), jnp.float32)
    for ky in range(3):
        for kx in range(3):
            sa = lax.slice(pa, (ky, kx), (ky + _H, kx + _W))
            sm = lax.slice(pm, (ky, kx), (ky + _H, kx + _W))
            acc = acc + w_ref[0, 0, ky, kx] * sa + w_ref[0, 1, ky, kx] * sm
    return 1.0 / (1.0 + jnp.exp(-acc))


def _maps_body(x_ref, imp_ref, w1_ref, w2_ref, im_ref, sub_ref, acc):
    t = pl.program_id(1)

    @pl.when(t == 0)
    def _init():
        acc[0] = jnp.zeros((_H, _W), jnp.float32)
        acc[1] = jnp.full((_H, _W), -jnp.inf, jnp.float32)
        acc[2] = jnp.zeros((_H, _W), jnp.float32)
        acc[3] = jnp.full((_H, _W), -jnp.inf, jnp.float32)

    xb = x_ref[0, 0]
    mrow = imp_ref[0, 0]
    mc = mrow[:, None, None]
    xi = xb * mc
    xs = xb - xi
    acc[0] += jnp.sum(xi, axis=0)
    acc[1] = jnp.maximum(acc[1], jnp.max(xi, axis=0))
    acc[2] += jnp.sum(xs, axis=0)
    acc[3] = jnp.maximum(acc[3], jnp.max(xs, axis=0))

    @pl.when(t == _T - 1)
    def _fin():
        scale = 1.0 / (_T * _C * 0.5)
        im_ref[0, 0] = _conv3x3_sigmoid(acc[0] * scale, acc[1], w1_ref)
        sub_ref[0, 0] = _conv3x3_sigmoid(acc[2] * scale, acc[3], w2_ref)


def _apply_body(x_ref, imp_ref, im_ref, sub_ref, o_ref):
    xb = x_ref[0, 0]
    mrow = imp_ref[0, 0][:, None, None]
    im = im_ref[0, 0][None]
    sub = sub_ref[0, 0][None]
    sel = mrow * im + (1.0 - mrow) * sub
    o_ref[0, 0] = xb * sel


def kernel(x, alpha, beta, fc_w, conv1_w, conv2_w):
    f32 = jnp.float32
    ab = jnp.concatenate([alpha, beta]).reshape(1, 2).astype(f32)

    m = pl.pallas_call(
        _stats_body,
        grid=(_N, _T),
        in_specs=[
            pl.BlockSpec((1, 1, _C, _H, _W), lambda i, j: (i, j, 0, 0, 0)),
            pl.BlockSpec((1, 2), lambda i, j: (0, 0)),
        ],
        out_specs=pl.BlockSpec((1, 1, 1, _C), lambda i, j: (i, j, 0, 0)),
        out_shape=jax.ShapeDtypeStruct((_N, _T, 1, _C), f32),
    )(x, ab)

    imp = pl.pallas_call(
        _mask_body,
        out_shape=jax.ShapeDtypeStruct((_T, 1, _C), f32),
    )(m, fc_w)

    im_map, sub_map = pl.pallas_call(
        _maps_body,
        grid=(_N, _T),
        in_specs=[
            pl.BlockSpec((1, 1, _C, _H, _W), lambda i, j: (i, j, 0, 0, 0)),
            pl.BlockSpec((1, 1, _C), lambda i, j: (j, 0, 0)),
            pl.BlockSpec((1, 2, 3, 3), lambda i, j: (0, 0, 0, 0)),
            pl.BlockSpec((1, 2, 3, 3), lambda i, j: (0, 0, 0, 0)),
        ],
        out_specs=[
            pl.BlockSpec((1, 1, _H, _W), lambda i, j: (i, 0, 0, 0)),
            pl.BlockSpec((1, 1, _H, _W), lambda i, j: (i, 0, 0, 0)),
        ],
        out_shape=[
            jax.ShapeDtypeStruct((_N, 1, _H, _W), f32),
            jax.ShapeDtypeStruct((_N, 1, _H, _W), f32),
        ],
        scratch_shapes=[pltpu.VMEM((4, _H, _W), f32)],
    )(x, imp, conv1_w, conv2_w)

    out = pl.pallas_call(
        _apply_body,
        grid=(_N, _T),
        in_specs=[
            pl.BlockSpec((1, 1, _C, _H, _W), lambda i, j: (i, j, 0, 0, 0)),
            pl.BlockSpec((1, 1, _C), lambda i, j: (j, 0, 0)),
            pl.BlockSpec((1, 1, _H, _W), lambda i, j: (i, 0, 0, 0)),
            pl.BlockSpec((1, 1, _H, _W), lambda i, j: (i, 0, 0, 0)),
        ],
        out_specs=pl.BlockSpec((1, 1, _C, _H, _W), lambda i, j: (i, j, 0, 0, 0)),
        out_shape=jax.ShapeDtypeStruct((_N, _T, _C, _H, _W), f32),
    )(x, imp, im_map, sub_map)
    return out


# speedup vs baseline: 4.0692x; 4.0692x over previous
import jax
import jax.numpy as jnp
from jax import lax
from jax.experimental import pallas as pl
from jax.experimental.pallas import tpu as pltpu

_N, _T, _C, _H, _W = 4, 16, 128, 32, 32
_KC, _KT = _C // 2, _T // 2
_HW = _H * _W


def _stats_body(x_ref, ab_ref, m_ref):
    a = ab_ref[0, 0]
    b = ab_ref[0, 1]
    ca = (0.5 + a) * (1.0 / _HW)
    cb = 0.5 + b
    rows = []
    for t in range(_T):
        xt = x_ref[0, t]
        rows.append(ca * jnp.sum(xt, axis=1) + cb * jnp.max(xt, axis=1))
    m_ref[0] = jnp.stack(rows, axis=0)


def _mask_body(m_ref, fc_ref, imp_ref):
    fc = fc_ref[...]
    jlt_c = (lax.broadcasted_iota(jnp.int32, (_C, _C), 1)
             < lax.broadcasted_iota(jnp.int32, (_C, _C), 0))
    jlt_t = (lax.broadcasted_iota(jnp.int32, (_T, _T), 1)
             < lax.broadcasted_iota(jnp.int32, (_T, _T), 0))
    chan = jnp.zeros((_T, _C), jnp.float32)
    ct = jnp.zeros((_C, _T), jnp.float32)
    for n in range(_N):
        mn = m_ref[n]
        m2 = jnp.dot(fc, mn, preferred_element_type=jnp.float32)
        vi = m2[:, :, None]
        vj = m2[:, None, :]
        g = (vj > vi) | ((vj == vi) & jlt_c[None])
        cnt = jnp.sum(g.astype(jnp.float32), axis=2)
        chan = jnp.maximum(chan, (cnt < _KC).astype(jnp.float32))
        mt = m2.T
        vi2 = mt[:, :, None]
        vj2 = mt[:, None, :]
        g2 = (vj2 > vi2) | ((vj2 == vi2) & jlt_t[None])
        cnt2 = jnp.sum(g2.astype(jnp.float32), axis=2)
        ct = jnp.maximum(ct, (cnt2 < _KT).astype(jnp.float32))
    ctt = ct.T
    imp = chan + ctt - 2.0 * chan * ctt
    imp_ref[...] = imp


def _shift_flat(img, s):
    if s == 0:
        return img
    if s > 0:
        z = jnp.zeros((1, s), jnp.float32)
        return jnp.concatenate([img[:, s:], z], axis=1)
    z = jnp.zeros((1, -s), jnp.float32)
    return jnp.concatenate([z, img[:, :_HW + s]], axis=1)


def _conv3x3_sigmoid_flat(avg_v, max_v, w_ref):
    col = lax.broadcasted_iota(jnp.int32, (1, _HW), 1) % _W
    m_lo = (col != 0).astype(jnp.float32)
    m_hi = (col != _W - 1).astype(jnp.float32)
    a2 = avg_v[None]
    m2 = max_v[None]
    acc = jnp.zeros((1, _HW), jnp.float32)
    for ky in range(3):
        for kx in range(3):
            s = (ky - 1) * _W + (kx - 1)
            term = w_ref[0, 0, ky, kx] * _shift_flat(a2, s) \
                 + w_ref[0, 1, ky, kx] * _shift_flat(m2, s)
            if kx == 0:
                term = term * m_lo
            elif kx == 2:
                term = term * m_hi
            acc = acc + term
    return 1.0 / (1.0 + jnp.exp(-acc))


def _fused_body(x_ref, imp_ref, w1_ref, w2_ref, o_ref):
    imp = imp_ref[...]
    im_sum = jnp.zeros((_HW,), jnp.float32)
    tot_sum = jnp.zeros((_HW,), jnp.float32)
    im_max = jnp.full((_HW,), -jnp.inf, jnp.float32)
    sub_max = jnp.full((_HW,), -jnp.inf, jnp.float32)
    for t in range(_T):
        xt = x_ref[0, t]
        mc = imp[t][:, None]
        xi = xt * mc
        im_sum += jnp.sum(xi, axis=0)
        tot_sum += jnp.sum(xt, axis=0)
        im_max = jnp.maximum(im_max, jnp.max(xi, axis=0))
        sub_max = jnp.maximum(sub_max, jnp.max(xt - xi, axis=0))
    sub_sum = tot_sum - im_sum
    scale = 1.0 / (_T * _C * 0.5)
    im_map = _conv3x3_sigmoid_flat(im_sum * scale, im_max, w1_ref)
    sub_map = _conv3x3_sigmoid_flat(sub_sum * scale, sub_max, w2_ref)
    diff = im_map - sub_map
    for t in range(_T):
        mc = imp[t][:, None]
        sel = sub_map + mc * diff
        o_ref[0, t] = x_ref[0, t] * sel


def kernel(x, alpha, beta, fc_w, conv1_w, conv2_w):
    f32 = jnp.float32
    xf = x.reshape(_N, _T, _C, _HW)
    ab = jnp.concatenate([alpha, beta]).reshape(1, 2).astype(f32)

    m = pl.pallas_call(
        _stats_body,
        grid=(_N,),
        in_specs=[
            pl.BlockSpec((1, _T, _C, _HW), lambda i: (i, 0, 0, 0)),
            pl.BlockSpec((1, 2), lambda i: (0, 0)),
        ],
        out_specs=pl.BlockSpec((1, _T, _C), lambda i: (i, 0, 0)),
        out_shape=jax.ShapeDtypeStruct((_N, _T, _C), f32),
    )(xf, ab)

    imp = pl.pallas_call(
        _mask_body,
        out_shape=jax.ShapeDtypeStruct((_T, _C), f32),
    )(m, fc_w)

    out = pl.pallas_call(
        _fused_body,
        grid=(_N,),
        in_specs=[
            pl.BlockSpec((1, _T, _C, _HW), lambda i: (i, 0, 0, 0)),
            pl.BlockSpec((_T, _C), lambda i: (0, 0)),
            pl.BlockSpec((1, 2, 3, 3), lambda i: (0, 0, 0, 0)),
            pl.BlockSpec((1, 2, 3, 3), lambda i: (0, 0, 0, 0)),
        ],
        out_specs=pl.BlockSpec((1, _T, _C, _HW), lambda i: (i, 0, 0, 0)),
        out_shape=jax.ShapeDtypeStruct((_N, _T, _C, _HW), f32),
    )(xf, imp, conv1_w, conv2_w)
    return out.reshape(_N, _T, _C, _H, _W)


# speedup vs baseline: 8.6687x; 2.1303x over previous
import jax
import jax.numpy as jnp
from jax import lax
from jax.experimental import pallas as pl
from jax.experimental.pallas import tpu as pltpu

_N, _T, _C, _H, _W = 4, 16, 128, 32, 32
_KC, _KT = _C // 2, _T // 2
_HW = _H * _W


def _stats_body(x_ref, ab_ref, m_ref):
    a = ab_ref[0, 0]
    b = ab_ref[0, 1]
    ca = (0.5 + a) * (1.0 / _HW)
    cb = 0.5 + b
    rows = []
    for t in range(_T):
        xt = x_ref[0, t]
        rows.append(ca * jnp.sum(xt, axis=0) + cb * jnp.max(xt, axis=0))
    m_ref[0] = jnp.stack(rows, axis=0)


def _mask_body(m_ref, fc_ref, imp_ref):
    fc = fc_ref[...]
    jlt_c = (lax.broadcasted_iota(jnp.int32, (_C, _C), 1)
             < lax.broadcasted_iota(jnp.int32, (_C, _C), 0))
    jlt_t = (lax.broadcasted_iota(jnp.int32, (_T, _T), 1)
             < lax.broadcasted_iota(jnp.int32, (_T, _T), 0))
    chan = jnp.zeros((_T, _C), jnp.float32)
    ct = jnp.zeros((_C, _T), jnp.float32)
    for n in range(_N):
        mn = m_ref[n]
        m2 = jnp.dot(fc, mn, preferred_element_type=jnp.float32)
        vi = m2[:, :, None]
        vj = m2[:, None, :]
        g = (vj > vi) | ((vj == vi) & jlt_c[None])
        cnt = jnp.sum(g.astype(jnp.float32), axis=2)
        chan = jnp.maximum(chan, (cnt < _KC).astype(jnp.float32))
        mt = m2.T
        vi2 = mt[:, :, None]
        vj2 = mt[:, None, :]
        g2 = (vj2 > vi2) | ((vj2 == vi2) & jlt_t[None])
        cnt2 = jnp.sum(g2.astype(jnp.float32), axis=2)
        ct = jnp.maximum(ct, (cnt2 < _KT).astype(jnp.float32))
    ctt = ct.T
    imp = chan + ctt - 2.0 * chan * ctt
    imp_ref[...] = imp


def _shift_col(col, s):
    if s == 0:
        return col
    if s > 0:
        z = jnp.zeros((s, 1), jnp.float32)
        return jnp.concatenate([col[s:], z], axis=0)
    z = jnp.zeros((-s, 1), jnp.float32)
    return jnp.concatenate([z, col[:_HW + s]], axis=0)


def _conv3x3_sigmoid_col(avg_c, max_c, w_ref):
    pix = lax.broadcasted_iota(jnp.int32, (_HW, 1), 0) % _W
    m_lo = (pix != 0).astype(jnp.float32)
    m_hi = (pix != _W - 1).astype(jnp.float32)
    acc = jnp.zeros((_HW, 1), jnp.float32)
    for ky in range(3):
        for kx in range(3):
            s = (ky - 1) * _W + (kx - 1)
            term = w_ref[0, 0, ky, kx] * _shift_col(avg_c, s) \
                 + w_ref[0, 1, ky, kx] * _shift_col(max_c, s)
            if kx == 0:
                term = term * m_lo
            elif kx == 2:
                term = term * m_hi
            acc = acc + term
    return 1.0 / (1.0 + jnp.exp(-acc))


def _fused_body(x_ref, imp_ref, w1_ref, w2_ref, o_ref):
    imp = imp_ref[...]
    acc_im = jnp.zeros((_HW, _C), jnp.float32)
    acc_tot = jnp.zeros((_HW, _C), jnp.float32)
    mx_im = jnp.full((_HW, _C), -jnp.inf, jnp.float32)
    mx_sub = jnp.full((_HW, _C), -jnp.inf, jnp.float32)
    for t in range(_T):
        xt = x_ref[0, t]
        xi = xt * imp[t][None, :]
        acc_im = acc_im + xi
        acc_tot = acc_tot + xt
        mx_im = jnp.maximum(mx_im, xi)
        mx_sub = jnp.maximum(mx_sub, xt - xi)
    scale = 1.0 / (_T * _C * 0.5)
    im_sum = jnp.sum(acc_im, axis=1, keepdims=True)
    tot_sum = jnp.sum(acc_tot, axis=1, keepdims=True)
    im_max = jnp.max(mx_im, axis=1, keepdims=True)
    sub_max = jnp.max(mx_sub, axis=1, keepdims=True)
    im_map = _conv3x3_sigmoid_col(im_sum * scale, im_max, w1_ref)
    sub_map = _conv3x3_sigmoid_col((tot_sum - im_sum) * scale, sub_max, w2_ref)
    diff = im_map - sub_map
    for t in range(_T):
        sel = sub_map + diff * imp[t][None, :]
        o_ref[0, t] = x_ref[0, t] * sel


def kernel(x, alpha, beta, fc_w, conv1_w, conv2_w):
    f32 = jnp.float32
    xr = x.transpose(0, 1, 3, 4, 2).reshape(_N, _T, _HW, _C)
    ab = jnp.concatenate([alpha, beta]).reshape(1, 2).astype(f32)

    m = pl.pallas_call(
        _stats_body,
        grid=(_N,),
        in_specs=[
            pl.BlockSpec((1, _T, _HW, _C), lambda i: (i, 0, 0, 0)),
            pl.BlockSpec((1, 2), lambda i: (0, 0)),
        ],
        out_specs=pl.BlockSpec((1, _T, _C), lambda i: (i, 0, 0)),
        out_shape=jax.ShapeDtypeStruct((_N, _T, _C), f32),
    )(xr, ab)

    imp = pl.pallas_call(
        _mask_body,
        out_shape=jax.ShapeDtypeStruct((_T, _C), f32),
    )(m, fc_w)

    out = pl.pallas_call(
        _fused_body,
        grid=(_N,),
        in_specs=[
            pl.BlockSpec((1, _T, _HW, _C), lambda i: (i, 0, 0, 0)),
            pl.BlockSpec((_T, _C), lambda i: (0, 0)),
            pl.BlockSpec((1, 2, 3, 3), lambda i: (0, 0, 0, 0)),
            pl.BlockSpec((1, 2, 3, 3), lambda i: (0, 0, 0, 0)),
        ],
        out_specs=pl.BlockSpec((1, _T, _HW, _C), lambda i: (i, 0, 0, 0)),
        out_shape=jax.ShapeDtypeStruct((_N, _T, _HW, _C), f32),
    )(xr, imp, conv1_w, conv2_w)
    return out.reshape(_N, _T, _H, _W, _C).transpose(0, 1, 4, 2, 3)
